# baseline (speedup 1.0000x reference)
import functools

import jax
import jax.numpy as jnp
from jax import lax
from jax.experimental import pallas as pl
from jax.experimental.pallas import tpu as pltpu, tpu_sc as plsc

B, S, D = 2, 2048, 128
N_TOK = B * S
N_RULES = 64
VOCAB = 32000
NW = 32
TOK_PER_W = N_TOK // NW


@functools.cache
def _make_rule_lookup():
    mesh = plsc.VectorSubcoreMesh(core_axis_name="c", subcore_axis_name="s")

    @functools.partial(
        pl.kernel,
        mesh=mesh,
        out_type=jax.ShapeDtypeStruct((N_TOK,), jnp.int32),
        scratch_types=[
            pltpu.VMEM((TOK_PER_W,), jnp.int32),
            pltpu.VMEM((TOK_PER_W,), jnp.int32),
            pltpu.SemaphoreType.DMA,
        ],
    )
    def _rule_lookup(ids_hbm, table_hbm, out_hbm, idx_v, rows_v, sem):
        wid = lax.axis_index("s") * 2 + lax.axis_index("c")
        base = wid * TOK_PER_W
        pltpu.sync_copy(ids_hbm.at[pl.ds(base, TOK_PER_W)], idx_v)
        pltpu.async_copy(table_hbm.at[idx_v], rows_v, sem).wait()
        pltpu.sync_copy(rows_v, out_hbm.at[pl.ds(base, TOK_PER_W)])

    return _rule_lookup


TBLK = 1024
NBLK = N_TOK // TBLK


def _tc_body(idx_ref, h_ref, rt_ref, o_ref, rtb_ref):
    i = pl.program_id(0)

    @pl.when(i == 0)
    def _():
        rt = rt_ref[...]
        sq = jnp.sum(rt * rt, axis=(1, 2))
        inv = 1.0 / jnp.maximum(jnp.sqrt(sq), 1e-12)
        rtb_ref[...] = (rt * inv[:, None, None]).astype(jnp.bfloat16)

    h = h_ref[...].astype(jnp.bfloat16)
    idx = idx_ref[0, 0, :]
    idxb = jnp.broadcast_to(idx[:, None], (TBLK, D))

    acc = jnp.zeros((TBLK, D), jnp.float32)
    for r in range(N_RULES):
        y = lax.dot_general(h, rtb_ref[r], (((1,), (0,)), ((), ())),
                            preferred_element_type=jnp.float32)
        acc = jnp.where(idxb == r, y, acc)
    o_ref[...] = acc


def _apply_rules(rule_idx, hidden, rule_transform):
    idx3 = rule_idx.reshape(NBLK, 1, TBLK)
    return pl.pallas_call(
        _tc_body,
        grid=(NBLK,),
        in_specs=[
            pl.BlockSpec((1, 1, TBLK), lambda i: (i, 0, 0)),
            pl.BlockSpec((TBLK, D), lambda i: (i, 0)),
            pl.BlockSpec((N_RULES, D, D), lambda i: (0, 0, 0)),
        ],
        out_specs=pl.BlockSpec((TBLK, D), lambda i: (i, 0)),
        out_shape=jax.ShapeDtypeStruct((N_TOK, D), jnp.float32),
        scratch_shapes=[pltpu.VMEM((N_RULES, D, D), jnp.bfloat16)],
    )(idx3, hidden, rule_transform)


def kernel(hidden_states, token_ids, token_rules, rule_transform):
    ids = token_ids.reshape(N_TOK).astype(jnp.int32)
    table = token_rules.astype(jnp.int32)
    rule_idx = _make_rule_lookup()(ids, table)
    h = hidden_states.reshape(N_TOK, D)
    out = _apply_rules(rule_idx, h, rule_transform)
    return out.reshape(B, S, D)


# speedup vs baseline: 1.0160x; 1.0160x over previous
import functools

import jax
import jax.numpy as jnp
from jax import lax
from jax.experimental import pallas as pl
from jax.experimental.pallas import tpu as pltpu, tpu_sc as plsc

B, S, D = 2, 2048, 128
N_TOK = B * S
N_RULES = 64
VOCAB = 32000
NW = 32
TOK_PER_W = N_TOK // NW


@functools.cache
def _make_rule_lookup():
    mesh = plsc.VectorSubcoreMesh(core_axis_name="c", subcore_axis_name="s")

    @functools.partial(
        pl.kernel,
        mesh=mesh,
        out_type=jax.ShapeDtypeStruct((N_TOK,), jnp.int32),
        scratch_types=[
            pltpu.VMEM((TOK_PER_W,), jnp.int32),
            pltpu.VMEM((TOK_PER_W,), jnp.int32),
            pltpu.SemaphoreType.DMA,
        ],
    )
    def _rule_lookup(ids_hbm, table_hbm, out_hbm, idx_v, rows_v, sem):
        wid = lax.axis_index("s") * 2 + lax.axis_index("c")
        base = wid * TOK_PER_W
        pltpu.sync_copy(ids_hbm.at[pl.ds(base, TOK_PER_W)], idx_v)
        pltpu.async_copy(table_hbm.at[idx_v], rows_v, sem).wait()
        pltpu.sync_copy(rows_v, out_hbm.at[pl.ds(base, TOK_PER_W)])

    return _rule_lookup


TBLK = 1024
NBLK = N_TOK // TBLK


def _tc_body(idx_ref, h_ref, rt_ref, o_ref, rtb_ref):
    i = pl.program_id(0)

    @pl.when(i == 0)
    def _():
        rt = rt_ref[...]
        sq = jnp.sum(rt * rt, axis=(1, 2))
        inv = 1.0 / jnp.maximum(jnp.sqrt(sq), 1e-12)
        rtb_ref[...] = (rt * inv[:, None, None]).astype(jnp.bfloat16)

    h = h_ref[...].astype(jnp.bfloat16)
    idx = idx_ref[0, 0, :]
    idxb = jnp.broadcast_to(
        idx.astype(jnp.bfloat16)[:, None], (TBLK, D))
    zero = jnp.zeros((TBLK, D), jnp.bfloat16)

    acc = jnp.zeros((TBLK, D), jnp.float32)
    for r in range(N_RULES):
        hm = jnp.where(idxb == jnp.bfloat16(r), h, zero)
        acc = acc + lax.dot_general(hm, rtb_ref[r],
                                    (((1,), (0,)), ((), ())),
                                    preferred_element_type=jnp.float32)
    o_ref[...] = acc


def _apply_rules(rule_idx, hidden, rule_transform):
    idx3 = rule_idx.reshape(NBLK, 1, TBLK)
    return pl.pallas_call(
        _tc_body,
        grid=(NBLK,),
        in_specs=[
            pl.BlockSpec((1, 1, TBLK), lambda i: (i, 0, 0)),
            pl.BlockSpec((TBLK, D), lambda i: (i, 0)),
            pl.BlockSpec((N_RULES, D, D), lambda i: (0, 0, 0)),
        ],
        out_specs=pl.BlockSpec((TBLK, D), lambda i: (i, 0)),
        out_shape=jax.ShapeDtypeStruct((N_TOK, D), jnp.float32),
        scratch_shapes=[pltpu.VMEM((N_RULES, D, D), jnp.bfloat16)],
    )(idx3, hidden, rule_transform)


def kernel(hidden_states, token_ids, token_rules, rule_transform):
    ids = token_ids.reshape(N_TOK).astype(jnp.int32)
    table = token_rules.astype(jnp.int32)
    rule_idx = _make_rule_lookup()(ids, table)
    h = hidden_states.reshape(N_TOK, D)
    out = _apply_rules(rule_idx, h, rule_transform)
    return out.reshape(B, S, D)
